# baseline (device time: 30579 ns/iter reference)
import jax
import jax.numpy as jnp
from jax import lax
from jax.experimental import pallas as pl
from jax.experimental.pallas import tpu as pltpu

B, S, H, Dh, Dr = 2, 256, 16, 64, 32
D = 1024
DC = 64
BS = B * S
BF = jnp.bfloat16
F32 = jnp.float32


def kernel(x, Wdkv, Wuk, Wuv, Wq, Wqr, Wkr, Wo):
    def body(x_ref, wdkv_ref, wuk_ref, wuv_ref, wq_ref, wqr_ref, wkr_ref,
             wo_ref, out_ref, c_send, c_recv, w_send, w_recv, attn_ref,
             wq_v, wqr_v, wo_v, send_sems, recv_sems, dma_sems):
        mx = lax.axis_index("x")
        my = lax.axis_index("y")
        mz = lax.axis_index("z")
        peer = (mx, my, 1 - mz)

        cp_wq = pltpu.make_async_copy(wq_ref, wq_v, dma_sems.at[0])
        cp_wqr = pltpu.make_async_copy(wqr_ref, wqr_v, dma_sems.at[1])
        cp_wo = pltpu.make_async_copy(wo_ref, wo_v, dma_sems.at[2])
        cp_wq.start()
        cp_wqr.start()
        cp_wo.start()

        barrier = pltpu.get_barrier_semaphore()
        pl.semaphore_signal(barrier, inc=1, device_id=peer,
                            device_id_type=pl.DeviceIdType.MESH)
        pl.semaphore_wait(barrier, 1)

        xb = x_ref[...].reshape(BS, D).astype(BF)

        c_send[...] = jnp.dot(xb, wdkv_ref[...].astype(BF),
                              preferred_element_type=F32).astype(BF)
        w_send[0, :, :] = wuk_ref[...].astype(BF)
        w_send[1, :, :] = wuv_ref[...].astype(BF)

        rdma_c = pltpu.make_async_remote_copy(
            src_ref=c_send, dst_ref=c_recv,
            send_sem=send_sems.at[0], recv_sem=recv_sems.at[0],
            device_id=peer, device_id_type=pl.DeviceIdType.MESH)
        rdma_w = pltpu.make_async_remote_copy(
            src_ref=w_send, dst_ref=w_recv,
            send_sem=send_sems.at[1], recv_sem=recv_sems.at[1],
            device_id=peer, device_id_type=pl.DeviceIdType.MESH)
        rdma_c.start()
        rdma_w.start()

        scale = (Dh + Dr) ** -0.5
        kr = jnp.dot(xb, wkr_ref[...].astype(BF),
                     preferred_element_type=F32).astype(BF)
        cp_wq.wait()
        q = (jnp.dot(xb, wq_v[...].astype(BF),
                     preferred_element_type=F32) * scale).astype(BF)
        cp_wqr.wait()
        qr = (jnp.dot(xb, wqr_v[...].astype(BF),
                      preferred_element_type=F32) * scale).astype(BF)

        rdma_c.wait()
        rdma_w.wait()

        c_mine = c_send[...]
        c_peer = c_recv[...]
        k = (jnp.dot(c_mine, w_send[0, :, :], preferred_element_type=F32)
             + jnp.dot(c_peer, w_recv[0, :, :],
                       preferred_element_type=F32)).astype(BF)
        v = (jnp.dot(c_mine, w_send[1, :, :], preferred_element_type=F32)
             + jnp.dot(c_peer, w_recv[1, :, :],
                       preferred_element_type=F32)).astype(BF)

        q3 = q.reshape(B, S, H * Dh)
        k3 = k.reshape(B, S, H * Dh)
        v3 = v.reshape(B, S, H * Dh)
        qr3 = qr.reshape(B, S, H * Dr)
        kr3 = kr.reshape(B, S, Dr)
        dn_qk = (((2,), (2,)), ((0,), (0,)))
        dn_pv = (((2,), (1,)), ((0,), (0,)))
        for h in range(H):
            q_h = q3[:, :, h * Dh:(h + 1) * Dh]
            k_h = k3[:, :, h * Dh:(h + 1) * Dh]
            qr_h = qr3[:, :, h * Dr:(h + 1) * Dr]
            s_h = (lax.dot_general(q_h, k_h, dn_qk,
                                   preferred_element_type=F32)
                   + lax.dot_general(qr_h, kr3, dn_qk,
                                     preferred_element_type=F32))
            e = jnp.exp(s_h)
            p = e.astype(BF)
            v_h = v3[:, :, h * Dh:(h + 1) * Dh]
            o_h = lax.dot_general(p, v_h, dn_pv, preferred_element_type=F32)
            o_h = o_h / jnp.sum(e, axis=2, keepdims=True)
            attn_ref[:, h * Dh:(h + 1) * Dh] = o_h.reshape(BS, Dh).astype(BF)

        cp_wo.wait()
        out = jnp.dot(attn_ref[...], wo_v[...].astype(BF),
                      preferred_element_type=F32)
        out_ref[...] = out.reshape(B, S, H * Dh)

    return pl.pallas_call(
        body,
        out_shape=jax.ShapeDtypeStruct((B, S, H * Dh), F32),
        in_specs=[
            pl.BlockSpec(memory_space=pltpu.VMEM),
            pl.BlockSpec(memory_space=pltpu.VMEM),
            pl.BlockSpec(memory_space=pltpu.VMEM),
            pl.BlockSpec(memory_space=pltpu.VMEM),
            pl.BlockSpec(memory_space=pl.ANY),
            pl.BlockSpec(memory_space=pl.ANY),
            pl.BlockSpec(memory_space=pltpu.VMEM),
            pl.BlockSpec(memory_space=pl.ANY),
        ],
        out_specs=pl.BlockSpec(memory_space=pltpu.VMEM),
        scratch_shapes=[
            pltpu.VMEM((BS, DC), BF),
            pltpu.VMEM((BS, DC), BF),
            pltpu.VMEM((2, DC, D), BF),
            pltpu.VMEM((2, DC, D), BF),
            pltpu.VMEM((BS, H * Dh), BF),
            pltpu.VMEM((D, D), F32),
            pltpu.VMEM((D, H * Dr), F32),
            pltpu.VMEM((D, D), F32),
            pltpu.SemaphoreType.DMA((2,)),
            pltpu.SemaphoreType.DMA((2,)),
            pltpu.SemaphoreType.DMA((3,)),
        ],
        compiler_params=pltpu.CompilerParams(collective_id=0),
    )(x, Wdkv, Wuk, Wuv, Wq, Wqr, Wkr, Wo)


# device time: 28567 ns/iter; 1.0704x vs baseline; 1.0704x over previous
import os

import jax
import jax.numpy as jnp
from jax import lax
from jax.experimental import pallas as pl
from jax.experimental.pallas import tpu as pltpu

B, S, H, Dh, Dr = 2, 256, 16, 64, 32
D = 1024
DC = 64
BS = B * S
HP = Dh + 2 * Dr
BF = jnp.bfloat16
F32 = jnp.float32
F8 = jnp.float8_e4m3fn
ABLATE = os.environ.get("ABLATE", "")


def kernel(x, Wdkv, Wuk, Wuv, Wq, Wqr, Wkr, Wo):
    def body(x_ref, wdkv_ref, wuk_ref, wuv_ref, wq_ref, wqr_ref, wkr_ref,
             wo_ref, out_ref, c_send, c_recv, w_send, w_recv, c_cat, wu_cat,
             qq_ref, kk_ref, attn_t, send_sems, recv_sems):
        mx = lax.axis_index("x")
        my = lax.axis_index("y")
        mz = lax.axis_index("z")
        peer = (mx, my, 1 - mz)

        barrier = pltpu.get_barrier_semaphore()
        pl.semaphore_signal(barrier, inc=1, device_id=peer,
                            device_id_type=pl.DeviceIdType.MESH)
        pl.semaphore_wait(barrier, 1)

        xb = x_ref[...].reshape(BS, D).astype(BF)

        w_send[0, :, :] = wuk_ref[...].astype(BF)
        w_send[1, :, :] = wuv_ref[...].astype(BF)
        rdma_w = pltpu.make_async_remote_copy(
            src_ref=w_send, dst_ref=w_recv,
            send_sem=send_sems.at[1], recv_sem=recv_sems.at[1],
            device_id=peer, device_id_type=pl.DeviceIdType.MESH)
        rdma_w.start()

        c = jnp.dot(xb, wdkv_ref[...].astype(BF),
                    preferred_element_type=F32).astype(BF)
        c_send[...] = c
        rdma_c = pltpu.make_async_remote_copy(
            src_ref=c_send, dst_ref=c_recv,
            send_sem=send_sems.at[0], recv_sem=recv_sems.at[0],
            device_id=peer, device_id_type=pl.DeviceIdType.MESH)
        rdma_c.start()

        scale = (Dh + Dr) ** -0.5
        if "qgemm" in ABLATE:
            kr = xb[:, :Dr]
            q = xb
            qr = xb[:, :H * Dr]
        else:
            kr = jnp.dot(xb, wkr_ref[...].astype(BF),
                         preferred_element_type=F32).astype(BF)
            q = (jnp.dot(xb, wq_ref[...].astype(BF),
                         preferred_element_type=F32) * scale).astype(BF)
            qr = (jnp.dot(xb, wqr_ref[...].astype(BF),
                          preferred_element_type=F32) * scale).astype(BF)

        zeros32 = jnp.zeros((BS, Dr), BF)
        for h in range(H):
            qq_ref[:, h * HP:h * HP + Dh] = q[:, h * Dh:(h + 1) * Dh]
            qq_ref[:, h * HP + Dh:h * HP + Dh + Dr] = qr[:, h * Dr:(h + 1) * Dr]
            qq_ref[:, h * HP + Dh + Dr:(h + 1) * HP] = zeros32

        rdma_c.wait()
        rdma_w.wait()

        c_cat[:, 0:DC] = c
        c_cat[:, DC:] = c_recv[...]
        wu_cat[0, 0:DC, :] = wuk_ref[...].astype(BF)
        wu_cat[0, DC:, :] = w_recv[0, :, :]
        wu_cat[1, 0:DC, :] = wuv_ref[...].astype(BF)
        wu_cat[1, DC:, :] = w_recv[1, :, :]
        cc = c_cat[...]
        k = jnp.dot(cc, wu_cat[0, :, :],
                    preferred_element_type=F32).astype(BF)
        v = jnp.dot(cc, wu_cat[1, :, :],
                    preferred_element_type=F32).astype(BF)

        for h in range(H):
            kk_ref[:, h * HP:h * HP + Dh] = k[:, h * Dh:(h + 1) * Dh]
            kk_ref[:, h * HP + Dh:h * HP + Dh + Dr] = kr
            kk_ref[:, h * HP + Dh + Dr:(h + 1) * HP] = kr

        qq3 = qq_ref[...].reshape(B, S, H * HP)
        kk3 = kk_ref[...].reshape(B, S, H * HP)
        v3 = v.reshape(B, S, H * Dh)
        dn_qk = (((2,), (2,)), ((0,), (0,)))
        dn_pv = (((2,), (1,)), ((0,), (0,)))
        for h in range(H if "attn" not in ABLATE else 0):
            s_h = lax.dot_general(qq3[:, :, h * HP:(h + 1) * HP],
                                  kk3[:, :, h * HP:(h + 1) * HP],
                                  dn_qk, preferred_element_type=F32)
            e = jnp.exp(s_h)
            p = e.astype(BF)
            o_t = lax.dot_general(v3[:, :, h * Dh:(h + 1) * Dh], p,
                                  (((1,), (2,)), ((0,), (0,))),
                                  preferred_element_type=F32)
            o_t = o_t / jnp.sum(e, axis=2)[:, None, :]
            ob = o_t.astype(BF)
            attn_t[h * Dh:(h + 1) * Dh, 0:S] = ob[0]
            attn_t[h * Dh:(h + 1) * Dh, S:BS] = ob[1]
        if "attn" in ABLATE:
            attn_t[...] = jnp.zeros((H * Dh, BS), BF)

        if "wo" in ABLATE:
            out_ref[...] = attn_t[...].astype(BF).reshape(B, S, H * Dh)
        else:
            out = lax.dot_general(attn_t[...], wo_ref[...].astype(BF),
                                  (((0,), (0,)), ((), ())),
                                  preferred_element_type=F32)
            out_ref[...] = out.astype(BF).reshape(B, S, H * Dh)

    return pl.pallas_call(
        body,
        out_shape=jax.ShapeDtypeStruct((B, S, H * Dh), BF),
        in_specs=[pl.BlockSpec(memory_space=pltpu.VMEM)] * 8,
        out_specs=pl.BlockSpec(memory_space=pltpu.VMEM),
        scratch_shapes=[
            pltpu.VMEM((BS, DC), BF),
            pltpu.VMEM((BS, DC), BF),
            pltpu.VMEM((2, DC, D), BF),
            pltpu.VMEM((2, DC, D), BF),
            pltpu.VMEM((BS, 2 * DC), BF),
            pltpu.VMEM((2, 2 * DC, D), BF),
            pltpu.VMEM((BS, H * HP), BF),
            pltpu.VMEM((BS, H * HP), BF),
            pltpu.VMEM((H * Dh, BS), BF),
            pltpu.SemaphoreType.DMA((2,)),
            pltpu.SemaphoreType.DMA((2,)),
        ],
        compiler_params=pltpu.CompilerParams(collective_id=0),
    )(x, Wdkv, Wuk, Wuv, Wq, Wqr, Wkr, Wo)


# device time: 27887 ns/iter; 1.0965x vs baseline; 1.0244x over previous
import os

import jax
import jax.numpy as jnp
from jax import lax
from jax.experimental import pallas as pl
from jax.experimental.pallas import tpu as pltpu

B, S, H, Dh, Dr = 2, 256, 16, 64, 32
D = 1024
DC = 64
BS = B * S
HP = Dh + 2 * Dr
BF = jnp.bfloat16
F32 = jnp.float32
F8 = jnp.float8_e4m3fn
ABLATE = os.environ.get("ABLATE", "")


def kernel(x, Wdkv, Wuk, Wuv, Wq, Wqr, Wkr, Wo):
    def body(x_ref, wdkv_ref, wuk_ref, wuv_ref, wq_ref, wqr_ref, wkr_ref,
             wo_ref, out_ref, c_send, c_recv, w_send, w_recv, c_cat, wu_cat,
             qq_ref, kk_ref, attn_t, send_sems, recv_sems):
        mx = lax.axis_index("x")
        my = lax.axis_index("y")
        mz = lax.axis_index("z")
        peer = (mx, my, 1 - mz)

        barrier = pltpu.get_barrier_semaphore()
        pl.semaphore_signal(barrier, inc=1, device_id=peer,
                            device_id_type=pl.DeviceIdType.MESH)
        pl.semaphore_wait(barrier, 1)

        xb = x_ref[...].reshape(BS, D).astype(BF)

        w_send[0, :, :] = wuk_ref[...].astype(BF)
        w_send[1, :, :] = wuv_ref[...].astype(BF)
        rdma_w = pltpu.make_async_remote_copy(
            src_ref=w_send, dst_ref=w_recv,
            send_sem=send_sems.at[1], recv_sem=recv_sems.at[1],
            device_id=peer, device_id_type=pl.DeviceIdType.MESH)
        rdma_w.start()

        c = jnp.dot(xb, wdkv_ref[...].astype(BF),
                    preferred_element_type=F32).astype(BF)
        c_send[...] = c
        rdma_c = pltpu.make_async_remote_copy(
            src_ref=c_send, dst_ref=c_recv,
            send_sem=send_sems.at[0], recv_sem=recv_sems.at[0],
            device_id=peer, device_id_type=pl.DeviceIdType.MESH)
        rdma_c.start()

        scale = (Dh + Dr) ** -0.5
        if "qgemm" in ABLATE:
            kr = xb[:, :Dr]
            q = xb
            qr = xb[:, :H * Dr]
        else:
            kr = jnp.dot(xb, wkr_ref[...].astype(BF),
                         preferred_element_type=F32).astype(BF)
            q = (jnp.dot(xb, wq_ref[...].astype(BF),
                         preferred_element_type=F32) * scale).astype(BF)
            qr = (jnp.dot(xb, wqr_ref[...].astype(BF),
                          preferred_element_type=F32) * scale).astype(BF)

        zeros32 = jnp.zeros((BS, Dr), BF)
        for h in range(H):
            qq_ref[:, h * HP:h * HP + Dh] = q[:, h * Dh:(h + 1) * Dh]
            qq_ref[:, h * HP + Dh:h * HP + Dh + Dr] = qr[:, h * Dr:(h + 1) * Dr]
            qq_ref[:, h * HP + Dh + Dr:(h + 1) * HP] = zeros32
            kk_ref[:, h * HP + Dh:h * HP + Dh + Dr] = kr
            kk_ref[:, h * HP + Dh + Dr:(h + 1) * HP] = kr
        c_cat[:, 0:DC] = c
        wu_cat[0, 0:DC, :] = wuk_ref[...].astype(BF)
        wu_cat[1, 0:DC, :] = wuv_ref[...].astype(BF)

        rdma_c.wait()
        rdma_w.wait()

        c_cat[:, DC:] = c_recv[...]
        wu_cat[0, DC:, :] = w_recv[0, :, :]
        wu_cat[1, DC:, :] = w_recv[1, :, :]
        cc = c_cat[...]
        k = jnp.dot(cc, wu_cat[0, :, :],
                    preferred_element_type=F32).astype(BF)
        v = jnp.dot(cc, wu_cat[1, :, :],
                    preferred_element_type=F32).astype(BF)

        for h in range(H):
            kk_ref[:, h * HP:h * HP + Dh] = k[:, h * Dh:(h + 1) * Dh]

        qq3 = qq_ref[...].reshape(B, S, H * HP)
        kk3 = kk_ref[...].reshape(B, S, H * HP)
        v3 = v.reshape(B, S, H * Dh)
        dn_qk = (((2,), (2,)), ((0,), (0,)))
        dn_pv = (((2,), (1,)), ((0,), (0,)))
        for h in range(H if "attn" not in ABLATE else 0):
            s_h = lax.dot_general(qq3[:, :, h * HP:(h + 1) * HP],
                                  kk3[:, :, h * HP:(h + 1) * HP],
                                  dn_qk, preferred_element_type=F32)
            e = jnp.exp(s_h)
            p = e.astype(BF)
            o_t = lax.dot_general(v3[:, :, h * Dh:(h + 1) * Dh], p,
                                  (((1,), (2,)), ((0,), (0,))),
                                  preferred_element_type=F32)
            o_t = o_t / jnp.sum(e, axis=2)[:, None, :]
            ob = o_t.astype(BF)
            attn_t[h * Dh:(h + 1) * Dh, 0:S] = ob[0]
            attn_t[h * Dh:(h + 1) * Dh, S:BS] = ob[1]
        if "attn" in ABLATE:
            attn_t[...] = jnp.zeros((H * Dh, BS), BF)

        if "wo" in ABLATE:
            out_ref[...] = attn_t[...].astype(BF).reshape(B, S, H * Dh)
        else:
            out = lax.dot_general(attn_t[...], wo_ref[...].astype(BF),
                                  (((0,), (0,)), ((), ())),
                                  preferred_element_type=F32)
            out_ref[...] = out.astype(BF).reshape(B, S, H * Dh)

    return pl.pallas_call(
        body,
        out_shape=jax.ShapeDtypeStruct((B, S, H * Dh), BF),
        in_specs=[pl.BlockSpec(memory_space=pltpu.VMEM)] * 8,
        out_specs=pl.BlockSpec(memory_space=pltpu.VMEM),
        scratch_shapes=[
            pltpu.VMEM((BS, DC), BF),
            pltpu.VMEM((BS, DC), BF),
            pltpu.VMEM((2, DC, D), BF),
            pltpu.VMEM((2, DC, D), BF),
            pltpu.VMEM((BS, 2 * DC), BF),
            pltpu.VMEM((2, 2 * DC, D), BF),
            pltpu.VMEM((BS, H * HP), BF),
            pltpu.VMEM((BS, H * HP), BF),
            pltpu.VMEM((H * Dh, BS), BF),
            pltpu.SemaphoreType.DMA((2,)),
            pltpu.SemaphoreType.DMA((2,)),
        ],
        compiler_params=pltpu.CompilerParams(collective_id=0),
    )(x, Wdkv, Wuk, Wuv, Wq, Wqr, Wkr, Wo)
